# baseline (speedup 1.0000x reference)
import functools

import jax
import jax.numpy as jnp
from jax import lax
from jax.experimental import pallas as pl
from jax.experimental.pallas import tpu as pltpu
from jax.experimental.pallas import tpu_sc as plsc

NUM_EXPERTS = 8
IN_DIM = 768
OUT_DIM = 768
B = 4096
T_LEN = 8

GBLK = 512
MBLK = 256
NBLK = B // MBLK + NUM_EXPERTS
PAD_B = NBLK * MBLK

SC_NC = 2
SC_NS = 16
SC_NW = SC_NC * SC_NS
CHUNK = B // SC_NW


def _gate_body(t_ref, Wg_ref, bg_ref, oh_ref):
    tm = jnp.mean(t_ref[...], axis=1)
    logits = lax.dot_general(
        tm, Wg_ref[...], (((1,), (1,)), ((), ())),
        preferred_element_type=jnp.float32) + bg_ref[...]
    mx = jnp.max(logits, axis=1, keepdims=True)
    eids = lax.broadcasted_iota(jnp.int32, logits.shape, 1)
    top1 = jnp.min(jnp.where(logits >= mx, eids, NUM_EXPERTS),
                   axis=1, keepdims=True)
    oh_ref[...] = (eids == top1).astype(jnp.int32)


def _gate_call(t, Wg, bg2):
    return pl.pallas_call(
        _gate_body,
        grid=(B // GBLK,),
        in_specs=[
            pl.BlockSpec((GBLK, T_LEN, OUT_DIM), lambda i: (i, 0, 0)),
            pl.BlockSpec((NUM_EXPERTS, OUT_DIM), lambda i: (0, 0)),
            pl.BlockSpec((1, NUM_EXPERTS), lambda i: (0, 0)),
        ],
        out_specs=pl.BlockSpec((GBLK, NUM_EXPERTS), lambda i: (i, 0)),
        out_shape=jax.ShapeDtypeStruct((B, NUM_EXPERTS), jnp.int32),
    )(t, Wg, bg2)


_MCH = 512


def _meta_body(oh_ref, pos_ref, bexp_ref, rank_ref):
    li = lax.broadcasted_iota(jnp.int32, (_MCH, _MCH), 0)
    lj = lax.broadcasted_iota(jnp.int32, (_MCH, _MCH), 1)
    Ltri = (lj < li).astype(jnp.float32)
    totals = jnp.zeros((1, NUM_EXPERTS), jnp.float32)
    for c in range(B // _MCH):
        oh = oh_ref[c * _MCH:(c + 1) * _MCH, :].astype(jnp.float32)
        rank_in = lax.dot_general(
            Ltri, oh, (((1,), (0,)), ((), ())),
            preferred_element_type=jnp.float32)
        rank_ref[c * _MCH:(c + 1) * _MCH, :] = rank_in + totals
        totals = totals + jnp.sum(oh, axis=0, keepdims=True)
    counts = totals.astype(jnp.int32)
    nb = (counts + (MBLK - 1)) // MBLK
    ia = lax.broadcasted_iota(jnp.int32, (NUM_EXPERTS, NUM_EXPERTS), 0)
    ib = lax.broadcasted_iota(jnp.int32, (NUM_EXPERTS, NUM_EXPERTS), 1)
    Uex = (ia < ib).astype(jnp.float32)
    exb = lax.dot_general(
        nb.astype(jnp.float32), Uex, (((1,), (0,)), ((), ())),
        preferred_element_type=jnp.float32)
    offsets = exb * float(MBLK)
    posf = rank_ref[...] + offsets
    pos_val = jnp.sum(oh_ref[...].astype(jnp.float32) * posf,
                      axis=1, keepdims=True)
    pos_ref[...] = jnp.broadcast_to(pos_val.astype(jnp.int32),
                                    (B, NUM_EXPERTS))
    cnb = exb + nb.astype(jnp.float32)
    jblk = lax.broadcasted_iota(jnp.int32, (8, 128), 1).astype(jnp.float32)
    bexp = jnp.zeros((8, 128), jnp.int32)
    for e in range(NUM_EXPERTS):
        ce = jnp.broadcast_to(cnb[:, e:e + 1], (8, 128))
        bexp = bexp + (jblk >= ce).astype(jnp.int32)
    bexp_ref[...] = jnp.minimum(bexp, NUM_EXPERTS - 1)


def _meta_call(oh):
    return pl.pallas_call(
        _meta_body,
        out_shape=(
            jax.ShapeDtypeStruct((B, NUM_EXPERTS), jnp.int32),
            jax.ShapeDtypeStruct((8, 128), jnp.int32),
        ),
        scratch_shapes=[pltpu.VMEM((B, NUM_EXPERTS), jnp.float32)],
    )(oh)


def _mm_body(be_ref, xs_ref, W_ref, b_ref, o_ref):
    j = pl.program_id(0)
    e = be_ref[j]
    w = W_ref[e]
    o_ref[...] = lax.dot_general(
        xs_ref[...], w, (((1,), (1,)), ((), ())),
        preferred_element_type=jnp.float32) + b_ref[e]


def _mm_call(be, xs, W, b3):
    grid_spec = pltpu.PrefetchScalarGridSpec(
        num_scalar_prefetch=1,
        grid=(NBLK,),
        in_specs=[
            pl.BlockSpec((MBLK, IN_DIM), lambda j, be_s: (j, 0)),
            pl.BlockSpec((NUM_EXPERTS, OUT_DIM, IN_DIM),
                         lambda j, be_s: (0, 0, 0)),
            pl.BlockSpec((NUM_EXPERTS, 1, OUT_DIM), lambda j, be_s: (0, 0, 0)),
        ],
        out_specs=pl.BlockSpec((MBLK, OUT_DIM), lambda j, be_s: (j, 0)),
    )
    return pl.pallas_call(
        _mm_body,
        grid_spec=grid_spec,
        out_shape=jax.ShapeDtypeStruct((PAD_B, OUT_DIM), jnp.float32),
    )(be, xs, W, b3)


@functools.lru_cache(maxsize=None)
def _sc_kernels():
    mesh = plsc.VectorSubcoreMesh(core_axis_name="c", subcore_axis_name="s")

    @functools.partial(
        pl.kernel,
        out_type=jax.ShapeDtypeStruct((PAD_B, IN_DIM), jnp.float32),
        mesh=mesh,
        scratch_types=[
            pltpu.VMEM((CHUNK,), jnp.int32),
            pltpu.VMEM((CHUNK, IN_DIM), jnp.float32),
            pltpu.SemaphoreType.DMA,
        ],
    )
    def _scatter_rows(x_hbm, pos_hbm, out_hbm, idx_v, rows_v, sem):
        wid = lax.axis_index("s") * SC_NC + lax.axis_index("c")
        base = wid * CHUNK
        pltpu.sync_copy(pos_hbm.at[pl.ds(base, CHUNK)], idx_v)
        pltpu.sync_copy(x_hbm.at[pl.ds(base, CHUNK)], rows_v)
        pltpu.async_copy(rows_v, out_hbm.at[idx_v], sem).wait()

    @functools.partial(
        pl.kernel,
        out_type=jax.ShapeDtypeStruct((B, OUT_DIM), jnp.float32),
        mesh=mesh,
        scratch_types=[
            pltpu.VMEM((CHUNK,), jnp.int32),
            pltpu.VMEM((CHUNK, OUT_DIM), jnp.float32),
            pltpu.SemaphoreType.DMA,
        ],
    )
    def _gather_rows(y_hbm, pos_hbm, out_hbm, idx_v, rows_v, sem):
        wid = lax.axis_index("s") * SC_NC + lax.axis_index("c")
        base = wid * CHUNK
        pltpu.sync_copy(pos_hbm.at[pl.ds(base, CHUNK)], idx_v)
        pltpu.async_copy(y_hbm.at[idx_v], rows_v, sem).wait()
        pltpu.sync_copy(rows_v, out_hbm.at[pl.ds(base, CHUNK)])

    return _scatter_rows, _gather_rows


def _scatter_call(x_flat, pos):
    return _sc_kernels()[0](x_flat, pos)


def _gather_call(ys, pos):
    return _sc_kernels()[1](ys, pos)


def kernel(x, t, W, b, Wg, bg):
    x_flat = jnp.squeeze(x, axis=1)
    bg2 = bg.reshape(1, NUM_EXPERTS)
    b3 = b.reshape(NUM_EXPERTS, 1, OUT_DIM)
    oh = _gate_call(t, Wg, bg2)
    pos8, bexp = _meta_call(oh)
    pos = pos8[:, 0]
    be = bexp[0, :NBLK]
    xs = _scatter_call(x_flat, pos)
    ys = _mm_call(be, xs, W, b3)
    out = _gather_call(ys, pos)
    return out.reshape(B, 1, OUT_DIM)


# speedup vs baseline: 1.0017x; 1.0017x over previous
import functools

import jax
import jax.numpy as jnp
from jax import lax
from jax.experimental import pallas as pl
from jax.experimental.pallas import tpu as pltpu
from jax.experimental.pallas import tpu_sc as plsc

NUM_EXPERTS = 8
IN_DIM = 768
OUT_DIM = 768
B = 4096
T_LEN = 8

GBLK = 512
MBLK = 256
NBLK = B // MBLK + NUM_EXPERTS
PAD_B = NBLK * MBLK

SC_NC = 2
SC_NS = 16
SC_NW = SC_NC * SC_NS
CHUNK = B // SC_NW


def _gate_body(t_ref, Wg_ref, bg_ref, oh_ref):
    tm = jnp.mean(t_ref[...], axis=1)
    logits = lax.dot_general(
        tm, Wg_ref[...], (((1,), (1,)), ((), ())),
        preferred_element_type=jnp.float32) + bg_ref[...]
    mx = jnp.max(logits, axis=1, keepdims=True)
    eids = lax.broadcasted_iota(jnp.int32, logits.shape, 1)
    top1 = jnp.min(jnp.where(logits >= mx, eids, NUM_EXPERTS),
                   axis=1, keepdims=True)
    oh_ref[...] = (eids == top1).astype(jnp.int32)


def _gate_call(t, Wg, bg2):
    return pl.pallas_call(
        _gate_body,
        grid=(B // GBLK,),
        in_specs=[
            pl.BlockSpec((GBLK, T_LEN, OUT_DIM), lambda i: (i, 0, 0)),
            pl.BlockSpec((NUM_EXPERTS, OUT_DIM), lambda i: (0, 0)),
            pl.BlockSpec((1, NUM_EXPERTS), lambda i: (0, 0)),
        ],
        out_specs=pl.BlockSpec((GBLK, NUM_EXPERTS), lambda i: (i, 0)),
        out_shape=jax.ShapeDtypeStruct((B, NUM_EXPERTS), jnp.int32),
    )(t, Wg, bg2)


_MCH = 512


def _meta_body(oh_ref, pos_ref, bexp_ref, rank_ref):
    li = lax.broadcasted_iota(jnp.int32, (_MCH, _MCH), 0)
    lj = lax.broadcasted_iota(jnp.int32, (_MCH, _MCH), 1)
    Ltri = (lj < li).astype(jnp.float32)
    totals = jnp.zeros((1, NUM_EXPERTS), jnp.float32)
    for c in range(B // _MCH):
        oh = oh_ref[c * _MCH:(c + 1) * _MCH, :].astype(jnp.float32)
        rank_in = lax.dot_general(
            Ltri, oh, (((1,), (0,)), ((), ())),
            preferred_element_type=jnp.float32)
        rank_ref[c * _MCH:(c + 1) * _MCH, :] = rank_in + totals
        totals = totals + jnp.sum(oh, axis=0, keepdims=True)
    counts = totals.astype(jnp.int32)
    nb = (counts + (MBLK - 1)) // MBLK
    ia = lax.broadcasted_iota(jnp.int32, (NUM_EXPERTS, NUM_EXPERTS), 0)
    ib = lax.broadcasted_iota(jnp.int32, (NUM_EXPERTS, NUM_EXPERTS), 1)
    Uex = (ia < ib).astype(jnp.float32)
    exb = lax.dot_general(
        nb.astype(jnp.float32), Uex, (((1,), (0,)), ((), ())),
        preferred_element_type=jnp.float32)
    offsets = exb * float(MBLK)
    posf = rank_ref[...] + offsets
    pos_val = jnp.sum(oh_ref[...].astype(jnp.float32) * posf,
                      axis=1, keepdims=True)
    pos_ref[...] = pos_val.astype(jnp.int32)
    cnb = exb + nb.astype(jnp.float32)
    jblk = lax.broadcasted_iota(jnp.int32, (1, 128), 1).astype(jnp.float32)
    bexp = jnp.zeros((1, 128), jnp.int32)
    for e in range(NUM_EXPERTS):
        ce = jnp.broadcast_to(cnb[:, e:e + 1], (1, 128))
        bexp = bexp + (jblk >= ce).astype(jnp.int32)
    bexp_ref[...] = jnp.minimum(bexp, NUM_EXPERTS - 1)


def _meta_call(oh):
    return pl.pallas_call(
        _meta_body,
        out_shape=(
            jax.ShapeDtypeStruct((B, 1), jnp.int32),
            jax.ShapeDtypeStruct((1, 128), jnp.int32),
        ),
        scratch_shapes=[pltpu.VMEM((B, NUM_EXPERTS), jnp.float32)],
    )(oh)


def _mm_body(be_ref, xs_ref, W_ref, b_ref, o_ref):
    j = pl.program_id(0)
    e = be_ref[0, j]
    w = W_ref[e]
    o_ref[...] = lax.dot_general(
        xs_ref[...], w, (((1,), (1,)), ((), ())),
        preferred_element_type=jnp.float32) + b_ref[e]


def _mm_call(be, xs, W, b3):
    grid_spec = pltpu.PrefetchScalarGridSpec(
        num_scalar_prefetch=1,
        grid=(NBLK,),
        in_specs=[
            pl.BlockSpec((MBLK, IN_DIM), lambda j, be_s: (j, 0)),
            pl.BlockSpec((NUM_EXPERTS, OUT_DIM, IN_DIM),
                         lambda j, be_s: (0, 0, 0)),
            pl.BlockSpec((NUM_EXPERTS, 1, OUT_DIM), lambda j, be_s: (0, 0, 0)),
        ],
        out_specs=pl.BlockSpec((MBLK, OUT_DIM), lambda j, be_s: (j, 0)),
    )
    return pl.pallas_call(
        _mm_body,
        grid_spec=grid_spec,
        out_shape=jax.ShapeDtypeStruct((PAD_B, OUT_DIM), jnp.float32),
    )(be, xs, W, b3)


@functools.lru_cache(maxsize=None)
def _sc_kernels():
    mesh = plsc.VectorSubcoreMesh(core_axis_name="c", subcore_axis_name="s")

    @functools.partial(
        pl.kernel,
        out_type=jax.ShapeDtypeStruct((PAD_B, IN_DIM), jnp.float32),
        mesh=mesh,
        scratch_types=[
            pltpu.VMEM((CHUNK,), jnp.int32),
            pltpu.VMEM((CHUNK, IN_DIM), jnp.float32),
            pltpu.SemaphoreType.DMA,
        ],
    )
    def _scatter_rows(x_hbm, pos_hbm, out_hbm, idx_v, rows_v, sem):
        wid = lax.axis_index("s") * SC_NC + lax.axis_index("c")
        base = wid * CHUNK
        pltpu.sync_copy(pos_hbm.at[pl.ds(base, CHUNK)], idx_v)
        pltpu.sync_copy(x_hbm.at[pl.ds(base, CHUNK)], rows_v)
        pltpu.async_copy(rows_v, out_hbm.at[idx_v], sem).wait()

    @functools.partial(
        pl.kernel,
        out_type=jax.ShapeDtypeStruct((B, OUT_DIM), jnp.float32),
        mesh=mesh,
        scratch_types=[
            pltpu.VMEM((CHUNK,), jnp.int32),
            pltpu.VMEM((CHUNK, OUT_DIM), jnp.float32),
            pltpu.SemaphoreType.DMA,
        ],
    )
    def _gather_rows(y_hbm, pos_hbm, out_hbm, idx_v, rows_v, sem):
        wid = lax.axis_index("s") * SC_NC + lax.axis_index("c")
        base = wid * CHUNK
        pltpu.sync_copy(pos_hbm.at[pl.ds(base, CHUNK)], idx_v)
        pltpu.async_copy(y_hbm.at[idx_v], rows_v, sem).wait()
        pltpu.sync_copy(rows_v, out_hbm.at[pl.ds(base, CHUNK)])

    return _scatter_rows, _gather_rows


def _scatter_call(x_flat, pos):
    return _sc_kernels()[0](x_flat, pos)


def _gather_call(ys, pos):
    return _sc_kernels()[1](ys, pos)


def kernel(x, t, W, b, Wg, bg):
    x_flat = jnp.squeeze(x, axis=1)
    bg2 = bg.reshape(1, NUM_EXPERTS)
    b3 = b.reshape(NUM_EXPERTS, 1, OUT_DIM)
    oh = _gate_call(t, Wg, bg2)
    pos2, bexp = _meta_call(oh)
    pos = pos2.reshape(B)
    xs = _scatter_call(x_flat, pos)
    ys = _mm_call(bexp, xs, W, b3)
    out = _gather_call(ys, pos)
    return out.reshape(B, 1, OUT_DIM)


# speedup vs baseline: 1.2243x; 1.2223x over previous
import functools

import jax
import jax.numpy as jnp
from jax import lax
from jax.experimental import pallas as pl
from jax.experimental.pallas import tpu as pltpu
from jax.experimental.pallas import tpu_sc as plsc

NUM_EXPERTS = 8
IN_DIM = 768
OUT_DIM = 768
B = 4096
T_LEN = 8

GBLK = 512
MBLK = 256
NBLK = B // MBLK + NUM_EXPERTS
PAD_B = NBLK * MBLK

SC_NC = 2
SC_NS = 16
SC_NW = SC_NC * SC_NS
CHUNK = B // SC_NW


def _gate_body(t_ref, Wg_ref, bg_ref, oh_ref):
    tm = jnp.mean(t_ref[...], axis=1)
    logits = lax.dot_general(
        tm, Wg_ref[...], (((1,), (1,)), ((), ())),
        preferred_element_type=jnp.float32) + bg_ref[...]
    mx = jnp.max(logits, axis=1, keepdims=True)
    eids = lax.broadcasted_iota(jnp.int32, logits.shape, 1)
    top1 = jnp.min(jnp.where(logits >= mx, eids, NUM_EXPERTS),
                   axis=1, keepdims=True)
    oh_ref[...] = (eids == top1).astype(jnp.int32)


def _gate_call(t, Wg, bg2):
    return pl.pallas_call(
        _gate_body,
        grid=(B // GBLK,),
        in_specs=[
            pl.BlockSpec((GBLK, T_LEN, OUT_DIM), lambda i: (i, 0, 0)),
            pl.BlockSpec((NUM_EXPERTS, OUT_DIM), lambda i: (0, 0)),
            pl.BlockSpec((1, NUM_EXPERTS), lambda i: (0, 0)),
        ],
        out_specs=pl.BlockSpec((GBLK, NUM_EXPERTS), lambda i: (i, 0)),
        out_shape=jax.ShapeDtypeStruct((B, NUM_EXPERTS), jnp.int32),
    )(t, Wg, bg2)


_MCH = 512


def _meta_body(oh_ref, pos_ref, bexp_ref, rank_ref):
    li = lax.broadcasted_iota(jnp.int32, (_MCH, _MCH), 0)
    lj = lax.broadcasted_iota(jnp.int32, (_MCH, _MCH), 1)
    Ltri = (lj < li).astype(jnp.float32)
    totals = jnp.zeros((1, NUM_EXPERTS), jnp.float32)
    for c in range(B // _MCH):
        oh = oh_ref[c * _MCH:(c + 1) * _MCH, :].astype(jnp.float32)
        rank_in = lax.dot_general(
            Ltri, oh, (((1,), (0,)), ((), ())),
            preferred_element_type=jnp.float32)
        rank_ref[c * _MCH:(c + 1) * _MCH, :] = rank_in + totals
        totals = totals + jnp.sum(oh, axis=0, keepdims=True)
    counts = totals.astype(jnp.int32)
    nb = (counts + (MBLK - 1)) // MBLK
    ia = lax.broadcasted_iota(jnp.int32, (NUM_EXPERTS, NUM_EXPERTS), 0)
    ib = lax.broadcasted_iota(jnp.int32, (NUM_EXPERTS, NUM_EXPERTS), 1)
    Uex = (ia < ib).astype(jnp.float32)
    exb = lax.dot_general(
        nb.astype(jnp.float32), Uex, (((1,), (0,)), ((), ())),
        preferred_element_type=jnp.float32)
    offsets = exb * float(MBLK)
    posf = rank_ref[...] + offsets
    pos_val = jnp.sum(oh_ref[...].astype(jnp.float32) * posf,
                      axis=1)
    pos_ref[...] = pos_val.astype(jnp.int32)
    cnb = exb + nb.astype(jnp.float32)
    jblk = lax.broadcasted_iota(jnp.int32, (1, 128), 1).astype(jnp.float32)
    bexp = jnp.zeros((1, 128), jnp.int32)
    for e in range(NUM_EXPERTS):
        ce = jnp.broadcast_to(cnb[:, e:e + 1], (1, 128))
        bexp = bexp + (jblk >= ce).astype(jnp.int32)
    bexp_ref[...] = bexp


def _meta_call(oh):
    return pl.pallas_call(
        _meta_body,
        out_shape=(
            jax.ShapeDtypeStruct((B,), jnp.int32),
            jax.ShapeDtypeStruct((1, 128), jnp.int32),
        ),
        scratch_shapes=[pltpu.VMEM((B, NUM_EXPERTS), jnp.float32)],
    )(oh)


def _mm_body(be_ref, xs_ref, W_ref, b_ref, o_ref):
    j = pl.program_id(0)
    e_raw = be_ref[0, j]
    e = jnp.minimum(e_raw, NUM_EXPERTS - 1)

    @pl.when(e_raw < NUM_EXPERTS)
    def _():
        w = W_ref[e]
        o_ref[...] = lax.dot_general(
            xs_ref[...], w, (((1,), (1,)), ((), ())),
            preferred_element_type=jnp.float32) + b_ref[e]


def _mm_call(be, xs, W, b3):
    grid_spec = pltpu.PrefetchScalarGridSpec(
        num_scalar_prefetch=1,
        grid=(NBLK,),
        in_specs=[
            pl.BlockSpec((MBLK, IN_DIM), lambda j, be_s: (j, 0)),
            pl.BlockSpec((NUM_EXPERTS, OUT_DIM, IN_DIM),
                         lambda j, be_s: (0, 0, 0)),
            pl.BlockSpec((NUM_EXPERTS, 1, OUT_DIM), lambda j, be_s: (0, 0, 0)),
        ],
        out_specs=pl.BlockSpec((MBLK, OUT_DIM), lambda j, be_s: (j, 0)),
    )
    return pl.pallas_call(
        _mm_body,
        grid_spec=grid_spec,
        out_shape=jax.ShapeDtypeStruct((PAD_B, OUT_DIM), jnp.float32),
    )(be, xs, W, b3)


@functools.lru_cache(maxsize=None)
def _sc_kernels():
    mesh = plsc.VectorSubcoreMesh(core_axis_name="c", subcore_axis_name="s")

    @functools.partial(
        pl.kernel,
        out_type=jax.ShapeDtypeStruct((PAD_B, IN_DIM), jnp.float32),
        mesh=mesh,
        scratch_types=[
            pltpu.VMEM((CHUNK,), jnp.int32),
            pltpu.VMEM((CHUNK, IN_DIM), jnp.float32),
            pltpu.SemaphoreType.DMA,
        ],
    )
    def _scatter_rows(x_hbm, pos_hbm, out_hbm, idx_v, rows_v, sem):
        wid = lax.axis_index("s") * SC_NC + lax.axis_index("c")
        base = wid * CHUNK
        pltpu.sync_copy(pos_hbm.at[pl.ds(base, CHUNK)], idx_v)
        pltpu.sync_copy(x_hbm.at[pl.ds(base, CHUNK), 0], rows_v)
        pltpu.async_copy(rows_v, out_hbm.at[idx_v], sem).wait()

    @functools.partial(
        pl.kernel,
        out_type=jax.ShapeDtypeStruct((B, 1, OUT_DIM), jnp.float32),
        mesh=mesh,
        scratch_types=[
            pltpu.VMEM((CHUNK,), jnp.int32),
            pltpu.VMEM((CHUNK, OUT_DIM), jnp.float32),
            pltpu.SemaphoreType.DMA,
        ],
    )
    def _gather_rows(y_hbm, pos_hbm, out_hbm, idx_v, rows_v, sem):
        wid = lax.axis_index("s") * SC_NC + lax.axis_index("c")
        base = wid * CHUNK
        pltpu.sync_copy(pos_hbm.at[pl.ds(base, CHUNK)], idx_v)
        pltpu.async_copy(y_hbm.at[idx_v], rows_v, sem).wait()
        pltpu.sync_copy(rows_v, out_hbm.at[pl.ds(base, CHUNK), 0])

    return _scatter_rows, _gather_rows


def _scatter_call(x_flat, pos):
    return _sc_kernels()[0](x_flat, pos)


def _gather_call(ys, pos):
    return _sc_kernels()[1](ys, pos)


def kernel(x, t, W, b, Wg, bg):
    bg2 = bg.reshape(1, NUM_EXPERTS)
    b3 = b.reshape(NUM_EXPERTS, 1, OUT_DIM)
    oh = _gate_call(t, Wg, bg2)
    pos, bexp = _meta_call(oh)
    xs = _scatter_call(x, pos)
    ys = _mm_call(bexp, xs, W, b3)
    return _gather_call(ys, pos)
